# baseline (speedup 1.0000x reference)
import functools

import jax
import jax.numpy as jnp
from jax import lax
from jax.experimental import pallas as pl
from jax.experimental.pallas import tpu as pltpu
from jax.experimental.pallas import tpu_sc as plsc

B = 4096
F = 26
H = 20
VOCAB = 100000
EMB = 16
FEAT = F * EMB
NW = 32
BPW = B // NW
NB = 1
NCHUNK = BPW // NB
RPC = NB * F
IPC = RPC * H
GLEN = 104
GSTREAMS = IPC // GLEN

ROWS_PER_TILE = VOCAB // 16


def _add_tables_t(t0t, t1t):
    def body(a_ref, b_ref, o_ref):
        o_ref[...] = a_ref[...] + b_ref[...]

    return pl.pallas_call(
        body,
        grid=(1,),
        in_specs=[
            pl.BlockSpec((EMB, VOCAB), lambda i: (0, 0)),
            pl.BlockSpec((EMB, VOCAB), lambda i: (0, 0)),
        ],
        out_specs=pl.BlockSpec((EMB, VOCAB), lambda i: (0, 0)),
        out_shape=jax.ShapeDtypeStruct((EMB, VOCAB), jnp.float32),
    )(t0t, t1t)


def _make_emb_sum():
    mesh = plsc.VectorSubcoreMesh(core_axis_name="c", subcore_axis_name="s")

    @functools.partial(
        pl.kernel,
        mesh=mesh,
        compiler_params=pltpu.CompilerParams(use_tc_tiling_on_sc=False),
        out_type=jax.ShapeDtypeStruct((B, FEAT), jnp.float32),
        scratch_types=[
            pltpu.VMEM((IPC,), jnp.int32),
            pltpu.VMEM((IPC,), jnp.int32),
            pltpu.VMEM((IPC, EMB), jnp.float32),
            pltpu.VMEM((IPC, EMB), jnp.float32),
            pltpu.VMEM((NB, FEAT), jnp.float32),
            pltpu.VMEM((NB, FEAT), jnp.float32),
            pltpu.VMEM_SHARED((VOCAB, EMB), jnp.float32),
            pltpu.SemaphoreType.DMA,
            pltpu.SemaphoreType.DMA,
            pltpu.SemaphoreType.DMA,
            pltpu.SemaphoreType.DMA,
            pltpu.SemaphoreType.DMA,
            pltpu.SemaphoreType.DMA,
        ],
    )
    def emb_sum(tsum_hbm, idx_hbm, out_hbm, idx0, idx1,
                rows0, rows1, acc0, acc1, tsum_sh,
                gsem0, gsem1, isem0, isem1, osem0, osem1):
        wid = lax.axis_index("s") * 2 + lax.axis_index("c")
        tid = lax.axis_index("s")
        idx_b = (idx0, idx1)
        rows_b = (rows0, rows1)
        acc_b = (acc0, acc1)
        gsem_b = (gsem0, gsem1)
        isem_b = (isem0, isem1)
        osem_b = (osem0, osem1)

        def fire_idx(c, slot):
            base = (wid * BPW + c * NB) * (F * H)
            pltpu.async_copy(idx_hbm.at[pl.ds(base, IPC)], idx_b[slot],
                             isem_b[slot])

        def wait_idx(c, slot):
            base = (wid * BPW + c * NB) * (F * H)
            pltpu.make_async_copy(idx_hbm.at[pl.ds(base, IPC)], idx_b[slot],
                                  isem_b[slot]).wait()

        fire_idx(0, 0)
        fire_idx(1, 1)

        tbase = tid * ROWS_PER_TILE
        pltpu.sync_copy(tsum_hbm.at[pl.ds(tbase, ROWS_PER_TILE)],
                        tsum_sh.at[pl.ds(tbase, ROWS_PER_TILE)])
        plsc.subcore_barrier()


        def fire_gathers(slot):
            for j in range(GSTREAMS):
                pltpu.async_copy(
                    tsum_sh.at[idx_b[slot].at[pl.ds(j * GLEN, GLEN)]],
                    rows_b[slot].at[pl.ds(j * GLEN, GLEN)],
                    gsem_b[slot],
                )

        def wait_gathers(slot):
            for j in range(GSTREAMS):
                pltpu.make_async_copy(
                    tsum_sh.at[idx_b[slot].at[pl.ds(j * GLEN, GLEN)]],
                    rows_b[slot].at[pl.ds(j * GLEN, GLEN)],
                    gsem_b[slot],
                ).wait()

        def out_copy(c, slot):
            return pltpu.make_async_copy(
                acc_b[slot], out_hbm.at[pl.ds(wid * BPW + c * NB, NB)],
                osem_b[slot])

        def sum_chunk(slot):
            rows_v = rows_b[slot]
            acc_v = acc_b[slot]
            for bi in range(NB):
                for f in range(F):
                    base = (bi * F + f) * H
                    v = [rows_v[base + h, :] for h in range(H)]
                    while len(v) > 1:
                        v = [v[i] + v[i + 1] for i in range(0, len(v) - 1, 2)] \
                            + ([v[-1]] if len(v) % 2 else [])
                    acc_v[bi, pl.ds(f * EMB, EMB)] = v[0]

        def step(c, slot):
            other = 1 - slot
            wait_gathers(slot)

            @pl.when(c + 2 < NCHUNK)
            def _():
                fire_idx(c + 2, slot)

            @pl.when(c + 1 < NCHUNK)
            def _():
                wait_idx(c + 1, other)
                fire_gathers(other)

            @pl.when(c >= 2)
            def _():
                out_copy(c - 2, slot).wait()

            sum_chunk(slot)
            out_copy(c, slot).start()

        wait_idx(0, 0)
        fire_gathers(0)

        def pair_body(g, carry):
            step(2 * g, 0)
            step(2 * g + 1, 1)
            return carry

        lax.fori_loop(0, NCHUNK // 2, pair_body, 0)
        out_copy(NCHUNK - 2, 0).wait()
        out_copy(NCHUNK - 1, 1).wait()

    return emb_sum


_emb_sum = _make_emb_sum()


def _mlp(s, gamma, beta, W1, b1, W2, b2, W3, b3):
    BB = 512

    def body(s_ref, g_ref, be_ref, w1_ref, b1_ref, w2_ref, b2_ref,
             w3_ref, b3_ref, o_ref):
        sb = s_ref[...]
        mean = jnp.mean(sb, axis=-1, keepdims=True)
        var = jnp.mean((sb - mean) ** 2, axis=-1, keepdims=True)
        hn = (sb - mean) * lax.rsqrt(var + 1e-5) * g_ref[...] + be_ref[...]
        h1 = jnp.maximum(jnp.dot(hn, w1_ref[...]) + b1_ref[...], 0.0)
        h2 = jnp.maximum(jnp.dot(h1, w2_ref[...]) + b2_ref[...], 0.0)
        o_ref[...] = jax.nn.sigmoid(jnp.dot(h2, w3_ref[...]) + b3_ref[...])

    full = lambda shape: pl.BlockSpec(shape, lambda i: tuple(0 for _ in shape))
    return pl.pallas_call(
        body,
        grid=(B // BB,),
        in_specs=[
            pl.BlockSpec((BB, FEAT), lambda i: (i, 0)),
            full((1, FEAT)),
            full((1, FEAT)),
            full((FEAT, 1024)),
            full((1, 1024)),
            full((1024, 512)),
            full((1, 512)),
            full((512, 1)),
            full((1, 1)),
        ],
        out_specs=pl.BlockSpec((BB, 1), lambda i: (i, 0)),
        out_shape=jax.ShapeDtypeStruct((B, 1), jnp.float32),
    )(s, gamma.reshape(1, FEAT), beta.reshape(1, FEAT), W1,
      b1.reshape(1, 1024), W2, b2.reshape(1, 512), W3, b3.reshape(1, 1))


def kernel(x, table0, table1, gamma, beta, W1, b1, W2, b2, W3, b3):
    idx = x.astype(jnp.int32).reshape(-1)
    tsum = _add_tables_t(table0.T, table1.T).T
    s = _emb_sum(tsum, idx)
    return _mlp(s, gamma, beta, W1, b1, W2, b2, W3, b3)


# speedup vs baseline: 1.3777x; 1.3777x over previous
import functools

import jax
import jax.numpy as jnp
from jax import lax
from jax.experimental import pallas as pl
from jax.experimental.pallas import tpu as pltpu
from jax.experimental.pallas import tpu_sc as plsc

B = 4096
F = 26
H = 20
VOCAB = 100000
EMB = 16
FEAT = F * EMB
NW = 32
BPW = B // NW
NB = 1
NCHUNK = BPW // NB
RPC = NB * F
IPC = RPC * H
GLEN = 104
GSTREAMS = IPC // GLEN

ROWS_PER_TILE = VOCAB // 16


def _add_tables_t(t0t, t1t):
    def body(a_ref, b_ref, o_ref):
        o_ref[...] = a_ref[...] + b_ref[...]

    return pl.pallas_call(
        body,
        grid=(1,),
        in_specs=[
            pl.BlockSpec((EMB, VOCAB), lambda i: (0, 0)),
            pl.BlockSpec((EMB, VOCAB), lambda i: (0, 0)),
        ],
        out_specs=pl.BlockSpec((EMB, VOCAB), lambda i: (0, 0)),
        out_shape=jax.ShapeDtypeStruct((EMB, VOCAB), jnp.float32),
    )(t0t, t1t)


def _make_emb_sum():
    mesh = plsc.VectorSubcoreMesh(core_axis_name="c", subcore_axis_name="s")

    @functools.partial(
        pl.kernel,
        mesh=mesh,
        compiler_params=pltpu.CompilerParams(use_tc_tiling_on_sc=False),
        out_type=jax.ShapeDtypeStruct((B, FEAT), jnp.float32),
        scratch_types=[
            pltpu.VMEM((IPC,), jnp.int32),
            pltpu.VMEM((IPC,), jnp.int32),
            pltpu.VMEM((IPC, EMB), jnp.float32),
            pltpu.VMEM((IPC, EMB), jnp.float32),
            pltpu.VMEM((NB, FEAT), jnp.float32),
            pltpu.VMEM((NB, FEAT), jnp.float32),
            pltpu.VMEM_SHARED((VOCAB, EMB), jnp.float32),
            pltpu.SemaphoreType.DMA,
            pltpu.SemaphoreType.DMA,
            pltpu.SemaphoreType.DMA,
            pltpu.SemaphoreType.DMA,
            pltpu.SemaphoreType.DMA,
            pltpu.SemaphoreType.DMA,
        ],
    )
    def emb_sum(tsum_hbm, idx_hbm, out_hbm, idx0, idx1,
                rows0, rows1, acc0, acc1, tsum_sh,
                gsem0, gsem1, isem0, isem1, osem0, osem1):
        wid = lax.axis_index("s") * 2 + lax.axis_index("c")
        tid = lax.axis_index("s")
        idx_b = (idx0, idx1)
        rows_b = (rows0, rows1)
        acc_b = (acc0, acc1)
        gsem_b = (gsem0, gsem1)
        isem_b = (isem0, isem1)
        osem_b = (osem0, osem1)

        def fire_idx(c, slot):
            base = (wid * BPW + c * NB) * (F * H)
            pltpu.async_copy(idx_hbm.at[pl.ds(base, IPC)], idx_b[slot],
                             isem_b[slot])

        def wait_idx(c, slot):
            base = (wid * BPW + c * NB) * (F * H)
            pltpu.make_async_copy(idx_hbm.at[pl.ds(base, IPC)], idx_b[slot],
                                  isem_b[slot]).wait()

        fire_idx(0, 0)
        fire_idx(1, 1)

        tbase = tid * ROWS_PER_TILE
        pltpu.sync_copy(tsum_hbm.at[pl.ds(tbase, ROWS_PER_TILE)],
                        tsum_sh.at[pl.ds(tbase, ROWS_PER_TILE)])
        plsc.subcore_barrier()


        def fire_gathers(slot):
            for j in range(GSTREAMS):
                pltpu.async_copy(
                    tsum_sh.at[idx_b[slot].at[pl.ds(j * GLEN, GLEN)]],
                    rows_b[slot].at[pl.ds(j * GLEN, GLEN)],
                    gsem_b[slot],
                )

        def wait_gathers(slot):
            for j in range(GSTREAMS):
                pltpu.make_async_copy(
                    tsum_sh.at[idx_b[slot].at[pl.ds(j * GLEN, GLEN)]],
                    rows_b[slot].at[pl.ds(j * GLEN, GLEN)],
                    gsem_b[slot],
                ).wait()

        def out_copy(c, slot):
            return pltpu.make_async_copy(
                acc_b[slot], out_hbm.at[pl.ds(wid * BPW + c * NB, NB)],
                osem_b[slot])

        def sum_chunk(slot):
            rows_v = rows_b[slot]
            acc_v = acc_b[slot]
            for bi in range(NB):

                def f_body(f, c2, bi=bi):
                    base = (bi * F + f) * H
                    v = [rows_v[base + h, :] for h in range(H)]
                    while len(v) > 1:
                        v = [v[i] + v[i + 1] for i in range(0, len(v) - 1, 2)] \
                            + ([v[-1]] if len(v) % 2 else [])
                    acc_v[bi, pl.ds(f * EMB, EMB)] = v[0]
                    return c2

                lax.fori_loop(0, F, f_body, 0)

        def step(c, slot):
            other = 1 - slot
            wait_gathers(slot)

            @pl.when(c + 2 < NCHUNK)
            def _():
                fire_idx(c + 2, slot)

            @pl.when(c + 1 < NCHUNK)
            def _():
                wait_idx(c + 1, other)
                fire_gathers(other)

            @pl.when(c >= 2)
            def _():
                out_copy(c - 2, slot).wait()

            sum_chunk(slot)
            out_copy(c, slot).start()

        wait_idx(0, 0)
        fire_gathers(0)

        def pair_body(g, carry):
            step(2 * g, 0)
            step(2 * g + 1, 1)
            return carry

        lax.fori_loop(0, NCHUNK // 2, pair_body, 0)
        out_copy(NCHUNK - 2, 0).wait()
        out_copy(NCHUNK - 1, 1).wait()

    return emb_sum


_emb_sum = _make_emb_sum()


def _mlp(s, gamma, beta, W1, b1, W2, b2, W3, b3):
    BB = 512

    def body(s_ref, g_ref, be_ref, w1_ref, b1_ref, w2_ref, b2_ref,
             w3_ref, b3_ref, o_ref):
        sb = s_ref[...]
        mean = jnp.mean(sb, axis=-1, keepdims=True)
        var = jnp.mean((sb - mean) ** 2, axis=-1, keepdims=True)
        hn = (sb - mean) * lax.rsqrt(var + 1e-5) * g_ref[...] + be_ref[...]
        h1 = jnp.maximum(jnp.dot(hn, w1_ref[...]) + b1_ref[...], 0.0)
        h2 = jnp.maximum(jnp.dot(h1, w2_ref[...]) + b2_ref[...], 0.0)
        o_ref[...] = jax.nn.sigmoid(jnp.dot(h2, w3_ref[...]) + b3_ref[...])

    full = lambda shape: pl.BlockSpec(shape, lambda i: tuple(0 for _ in shape))
    return pl.pallas_call(
        body,
        grid=(B // BB,),
        in_specs=[
            pl.BlockSpec((BB, FEAT), lambda i: (i, 0)),
            full((1, FEAT)),
            full((1, FEAT)),
            full((FEAT, 1024)),
            full((1, 1024)),
            full((1024, 512)),
            full((1, 512)),
            full((512, 1)),
            full((1, 1)),
        ],
        out_specs=pl.BlockSpec((BB, 1), lambda i: (i, 0)),
        out_shape=jax.ShapeDtypeStruct((B, 1), jnp.float32),
    )(s, gamma.reshape(1, FEAT), beta.reshape(1, FEAT), W1,
      b1.reshape(1, 1024), W2, b2.reshape(1, 512), W3, b3.reshape(1, 1))


def kernel(x, table0, table1, gamma, beta, W1, b1, W2, b2, W3, b3):
    idx = x.astype(jnp.int32).reshape(-1)
    tsum = _add_tables_t(table0.T, table1.T).T
    s = _emb_sum(tsum, idx)
    return _mlp(s, gamma, beta, W1, b1, W2, b2, W3, b3)
